# baseline (speedup 1.0000x reference)
import functools

import jax
import jax.numpy as jnp
from jax import lax
from jax.experimental import pallas as pl
from jax.experimental.pallas import tpu as pltpu
from jax.experimental.pallas import tpu_sc as plsc

N_NODES = 100000
D_FEAT = 128
NUM_GRAPHS = 512
HIDDEN = 256

NC = 2
NS = 16
NW = NC * NS
ROWS_W = 3128
CHUNK = 184
NCHUNKS = 17
NSLICE = D_FEAT // 16


def _sc_body(feat_hbm, ids_hbm, out_hbm, cnt_hbm, rowbuf, idsbuf, acc, cnt):
    cid = lax.axis_index("c")
    sid = lax.axis_index("s")
    wid = sid * NC + cid
    base = wid * ROWS_W
    end = jnp.minimum(base + ROWS_W, N_NODES)

    zero16 = jnp.zeros((16,), jnp.float32)

    def zacc(i, carry):
        acc[pl.ds(i * 16, 16)] = zero16
        return carry

    lax.fori_loop(0, NUM_GRAPHS * NSLICE, zacc, 0)

    def zcnt(i, carry):
        cnt[pl.ds(i * 16, 16)] = zero16
        return carry

    lax.fori_loop(0, NUM_GRAPHS, zcnt, 0)

    def row_body(i, carry):
        prev_s, accs, cntv = carry
        sseg = idsbuf[pl.ds(i, 16)][0]
        changed = sseg != prev_s

        @pl.when(changed)
        def _flush():
            tgt = jnp.maximum(prev_s, 0)
            for j in range(NSLICE):
                acc[pl.ds(tgt * D_FEAT + j * 16, 16)] = accs[j]
            cnt[pl.ds(tgt * 16, 16)] = cntv

        keep = jnp.broadcast_to(
            jnp.where(changed, jnp.float32(0.0), jnp.float32(1.0)), (16,))
        new_accs = tuple(
            accs[j] * keep + rowbuf[pl.ds(i * D_FEAT + j * 16, 16)]
            for j in range(NSLICE))
        new_cntv = cntv * keep + 1.0
        return sseg, new_accs, new_cntv

    def chunk_body(k, carry):
        p, prev_s, accs, cntv = carry
        b = jnp.minimum(base + k * CHUNK, end - CHUNK)
        pltpu.sync_copy(feat_hbm.at[pl.ds(b * D_FEAT, CHUNK * D_FEAT)], rowbuf)
        pltpu.sync_copy(ids_hbm.at[pl.ds(b, CHUNK)],
                        idsbuf.at[pl.ds(0, CHUNK)])
        skip = p - b
        prev_s, accs, cntv = lax.fori_loop(
            skip, CHUNK, row_body, (prev_s, accs, cntv))
        return b + CHUNK, prev_s, accs, cntv

    init = (base, jnp.int32(-1), tuple(zero16 for _ in range(NSLICE)), zero16)
    _, last_s, accs, cntv = lax.fori_loop(0, NCHUNKS, chunk_body, init)

    tgt = jnp.maximum(last_s, 0)
    for j in range(NSLICE):
        acc[pl.ds(tgt * D_FEAT + j * 16, 16)] = accs[j]
    cnt[pl.ds(tgt * 16, 16)] = cntv

    pltpu.sync_copy(acc, out_hbm.at[wid])
    pltpu.sync_copy(cnt, cnt_hbm.at[wid])


_sc_seg_sum = functools.partial(
    pl.kernel,
    out_type=[
        jax.ShapeDtypeStruct((NW, NUM_GRAPHS * D_FEAT), jnp.float32),
        jax.ShapeDtypeStruct((NW, NUM_GRAPHS * 16), jnp.float32),
    ],
    mesh=plsc.VectorSubcoreMesh(
        core_axis_name="c", subcore_axis_name="s",
        num_cores=NC, num_subcores=NS),
    scratch_types=[
        pltpu.VMEM((CHUNK * D_FEAT,), jnp.float32),
        pltpu.VMEM((CHUNK + 16,), jnp.int32),
        pltpu.VMEM((NUM_GRAPHS * D_FEAT,), jnp.float32),
        pltpu.VMEM((NUM_GRAPHS * 16,), jnp.float32),
    ],
)(_sc_body)


def _tc_body(p_ref, c_ref, W1_ref, b1_ref, W2_ref, b2_ref, W3_ref, b3_ref,
             out_ref):
    sums = p_ref[0]
    cnts = c_ref[0]
    for w in range(1, NW):
        sums = sums + p_ref[w]
        cnts = cnts + c_ref[w]
    pooled = sums / jnp.maximum(cnts[:, 0:1], 1.0)
    h = jnp.maximum(
        jnp.dot(pooled, W1_ref[...], preferred_element_type=jnp.float32)
        + b1_ref[...], 0.0)
    h = jnp.maximum(
        jnp.dot(h, W2_ref[...], preferred_element_type=jnp.float32)
        + b2_ref[...], 0.0)
    out_ref[...] = (
        jnp.dot(h, W3_ref[...], preferred_element_type=jnp.float32)
        + b3_ref[...])


def kernel(feat, segment_ids, W1, b1, W2, b2, W3, b3):
    ids = segment_ids.astype(jnp.int32)
    partials, counts = _sc_seg_sum(feat.reshape(N_NODES * D_FEAT), ids)
    pred = pl.pallas_call(
        _tc_body,
        out_shape=jax.ShapeDtypeStruct((NUM_GRAPHS, 1), jnp.float32),
    )(partials.reshape(NW, NUM_GRAPHS, D_FEAT),
      counts.reshape(NW, NUM_GRAPHS, 16),
      W1, b1.reshape(1, HIDDEN), W2, b2.reshape(1, HIDDEN),
      W3, b3.reshape(1, 1))
    return pred.reshape(NUM_GRAPHS)


# speedup vs baseline: 1.4507x; 1.4507x over previous
import functools

import jax
import jax.numpy as jnp
from jax import lax
from jax.experimental import pallas as pl
from jax.experimental.pallas import tpu as pltpu
from jax.experimental.pallas import tpu_sc as plsc

N_NODES = 100000
D_FEAT = 128
NUM_GRAPHS = 512
HIDDEN = 256

NC = 2
NS = 16
NW = NC * NS
ROWS_W = 3136
CHUNK = 192
NCHUNKS = 17
NGROUPS = CHUNK // 16
NSLICE = D_FEAT // 16

_GDN = lax.GatherDimensionNumbers(
    offset_dims=(), collapsed_slice_dims=(0,), start_index_map=(0,))


def _lane_bcast(v, r):
    idx = jnp.full((16,), r, dtype=jnp.int32)
    return lax.gather(v, idx[:, None], _GDN, (1,),
                      mode=lax.GatherScatterMode.PROMISE_IN_BOUNDS)


def _sc_body(feat_hbm, ids_hbm, out_hbm, cnt_hbm, rowbuf, idsbuf, acc, cnt):
    cid = lax.axis_index("c")
    sid = lax.axis_index("s")
    wid = sid * NC + cid
    base = wid * ROWS_W
    end = jnp.minimum(base + ROWS_W, N_NODES)

    zero16 = jnp.zeros((16,), jnp.float32)
    ones16 = jnp.ones((16,), jnp.float32)
    iota16 = lax.iota(jnp.int32, 16)

    def zrow(i, carry):
        for j in range(NSLICE):
            acc[pl.ds(i * D_FEAT + j * 16, 16)] = zero16
        cnt[pl.ds(i * 16, 16)] = zero16
        return carry

    lax.fori_loop(0, NUM_GRAPHS, zrow, 0)

    def group_body(g, carry):
        idv = idsbuf[pl.ds(g * 16, 16)]
        plsc.addupdate_scatter(cnt, [idv * 16 + iota16], ones16)
        prev = None
        for r in range(16):
            seg = _lane_bcast(idv, r)
            abase = seg * D_FEAT + iota16
            row_off = (g * 16 + r) * D_FEAT
            xs = [rowbuf[pl.ds(row_off + j * 16, 16)] for j in range(NSLICE)]
            if prev is not None:
                pabase, pxs = prev
                for j in range(NSLICE):
                    plsc.addupdate_scatter(acc, [pabase + j * 16], pxs[j])
            prev = (abase, xs)
        pabase, pxs = prev
        for j in range(NSLICE):
            plsc.addupdate_scatter(acc, [pabase + j * 16], pxs[j])
        return carry

    def chunk_body(k, p):
        b = jnp.minimum(base + k * CHUNK, end - CHUNK)
        pltpu.sync_copy(feat_hbm.at[pl.ds(b * D_FEAT, CHUNK * D_FEAT)], rowbuf)
        pltpu.sync_copy(ids_hbm.at[pl.ds(b, CHUNK)], idsbuf)
        gs = (p - b) // 16
        lax.fori_loop(gs, NGROUPS, group_body, 0)
        return b + CHUNK

    lax.fori_loop(0, NCHUNKS, chunk_body, base)

    pltpu.sync_copy(acc, out_hbm.at[wid])
    pltpu.sync_copy(cnt, cnt_hbm.at[wid])


_sc_seg_sum = functools.partial(
    pl.kernel,
    out_type=[
        jax.ShapeDtypeStruct((NW, NUM_GRAPHS * D_FEAT), jnp.float32),
        jax.ShapeDtypeStruct((NW, NUM_GRAPHS * 16), jnp.float32),
    ],
    mesh=plsc.VectorSubcoreMesh(
        core_axis_name="c", subcore_axis_name="s",
        num_cores=NC, num_subcores=NS),
    compiler_params=pltpu.CompilerParams(needs_layout_passes=False),
    scratch_types=[
        pltpu.VMEM((CHUNK * D_FEAT,), jnp.float32),
        pltpu.VMEM((CHUNK,), jnp.int32),
        pltpu.VMEM((NUM_GRAPHS * D_FEAT,), jnp.float32),
        pltpu.VMEM((NUM_GRAPHS * 16,), jnp.float32),
    ],
)(_sc_body)


def _tc_body(p_ref, c_ref, W1_ref, b1_ref, W2_ref, b2_ref, W3_ref, b3_ref,
             out_ref):
    sums = p_ref[0]
    cnts = c_ref[0]
    for w in range(1, NW):
        sums = sums + p_ref[w]
        cnts = cnts + c_ref[w]
    pooled = sums / jnp.maximum(
        jnp.sum(cnts, axis=1, keepdims=True), 1.0)
    h = jnp.maximum(
        jnp.dot(pooled, W1_ref[...], preferred_element_type=jnp.float32)
        + b1_ref[...], 0.0)
    h = jnp.maximum(
        jnp.dot(h, W2_ref[...], preferred_element_type=jnp.float32)
        + b2_ref[...], 0.0)
    out_ref[...] = (
        jnp.dot(h, W3_ref[...], preferred_element_type=jnp.float32)
        + b3_ref[...])


def kernel(feat, segment_ids, W1, b1, W2, b2, W3, b3):
    ids = segment_ids.astype(jnp.int32)
    partials, counts = _sc_seg_sum(feat.reshape(N_NODES * D_FEAT), ids)
    pred = pl.pallas_call(
        _tc_body,
        out_shape=jax.ShapeDtypeStruct((NUM_GRAPHS, 1), jnp.float32),
    )(partials.reshape(NW, NUM_GRAPHS, D_FEAT),
      counts.reshape(NW, NUM_GRAPHS, 16),
      W1, b1.reshape(1, HIDDEN), W2, b2.reshape(1, HIDDEN),
      W3, b3.reshape(1, 1))
    return pred.reshape(NUM_GRAPHS)


# speedup vs baseline: 2.0619x; 1.4213x over previous
import functools

import jax
import jax.numpy as jnp
from jax import lax
from jax.experimental import pallas as pl
from jax.experimental.pallas import tpu as pltpu
from jax.experimental.pallas import tpu_sc as plsc

N_NODES = 100000
D_FEAT = 128
NUM_GRAPHS = 512
HIDDEN = 256

NC = 2
NS = 16
NW = NC * NS
ROWS_W = 3136
CHUNK = 192
NPAIRS = 9
NGROUPS = CHUNK // 16
NSLICE = D_FEAT // 16

_GDN = lax.GatherDimensionNumbers(
    offset_dims=(), collapsed_slice_dims=(0,), start_index_map=(0,))


def _lane_bcast(v, r):
    idx = jnp.full((16,), r, dtype=jnp.int32)
    return lax.gather(v, idx[:, None], _GDN, (1,),
                      mode=lax.GatherScatterMode.PROMISE_IN_BOUNDS)


def _sc_body(feat_hbm, ids_hbm, out_hbm, cnt_hbm,
             rowA, idsA, rowB, idsB, acc, cnt, semA, semB):
    cid = lax.axis_index("c")
    sid = lax.axis_index("s")
    wid = sid * NC + cid
    base = wid * ROWS_W
    end = jnp.minimum(base + ROWS_W, N_NODES)

    zero16 = jnp.zeros((16,), jnp.float32)
    ones16 = jnp.ones((16,), jnp.float32)
    iota16 = lax.iota(jnp.int32, 16)

    def start_chunk(k, rowbuf, idsbuf, sem):
        b = jnp.minimum(base + k * CHUNK, end - CHUNK)
        pltpu.async_copy(
            feat_hbm.at[pl.ds(b * D_FEAT, CHUNK * D_FEAT)], rowbuf, sem)
        pltpu.async_copy(ids_hbm.at[pl.ds(b, CHUNK)], idsbuf, sem)

    def wait_chunk(rowbuf, idsbuf, sem):
        pltpu.make_async_copy(
            feat_hbm.at[pl.ds(0, CHUNK * D_FEAT)], rowbuf, sem).wait()
        pltpu.make_async_copy(
            ids_hbm.at[pl.ds(0, CHUNK)], idsbuf, sem).wait()

    start_chunk(0, rowA, idsA, semA)

    def zrow(i, carry):
        for j in range(NSLICE):
            acc[pl.ds(i * D_FEAT + j * 16, 16)] = zero16
        cnt[pl.ds(i * 16, 16)] = zero16
        return carry

    lax.fori_loop(0, NUM_GRAPHS, zrow, 0)

    def make_group_body(rowbuf, idsbuf):
        def group_body(g, carry):
            idv = idsbuf[pl.ds(g * 16, 16)]
            plsc.addupdate_scatter(cnt, [idv * 16 + iota16], ones16)
            prev = None
            for r in range(16):
                seg = _lane_bcast(idv, r)
                abase = seg * D_FEAT + iota16
                row_off = (g * 16 + r) * D_FEAT
                xs = [rowbuf[pl.ds(row_off + j * 16, 16)]
                      for j in range(NSLICE)]
                if prev is not None:
                    pabase, pxs = prev
                    for j in range(NSLICE):
                        plsc.addupdate_scatter(acc, [pabase + j * 16], pxs[j])
                prev = (abase, xs)
            pabase, pxs = prev
            for j in range(NSLICE):
                plsc.addupdate_scatter(acc, [pabase + j * 16], pxs[j])
            return carry
        return group_body

    def process(k, p, rowbuf, idsbuf):
        b = jnp.minimum(base + k * CHUNK, end - CHUNK)
        gs = (p - b) // 16
        lax.fori_loop(gs, NGROUPS, make_group_body(rowbuf, idsbuf), 0)
        return b + CHUNK

    def pair_body(t, p):
        k0 = 2 * t
        start_chunk(k0 + 1, rowB, idsB, semB)
        wait_chunk(rowA, idsA, semA)
        p = process(k0, p, rowA, idsA)

        @pl.when(t < NPAIRS - 1)
        def _():
            start_chunk(k0 + 2, rowA, idsA, semA)

        wait_chunk(rowB, idsB, semB)
        return process(k0 + 1, p, rowB, idsB)

    lax.fori_loop(0, NPAIRS, pair_body, base)

    pltpu.sync_copy(acc, out_hbm.at[wid])
    pltpu.sync_copy(cnt, cnt_hbm.at[wid])


_sc_seg_sum = functools.partial(
    pl.kernel,
    out_type=[
        jax.ShapeDtypeStruct((NW, NUM_GRAPHS * D_FEAT), jnp.float32),
        jax.ShapeDtypeStruct((NW, NUM_GRAPHS * 16), jnp.float32),
    ],
    mesh=plsc.VectorSubcoreMesh(
        core_axis_name="c", subcore_axis_name="s",
        num_cores=NC, num_subcores=NS),
    compiler_params=pltpu.CompilerParams(needs_layout_passes=False),
    scratch_types=[
        pltpu.VMEM((CHUNK * D_FEAT,), jnp.float32),
        pltpu.VMEM((CHUNK,), jnp.int32),
        pltpu.VMEM((CHUNK * D_FEAT,), jnp.float32),
        pltpu.VMEM((CHUNK,), jnp.int32),
        pltpu.VMEM((NUM_GRAPHS * D_FEAT,), jnp.float32),
        pltpu.VMEM((NUM_GRAPHS * 16,), jnp.float32),
        pltpu.SemaphoreType.DMA,
        pltpu.SemaphoreType.DMA,
    ],
)(_sc_body)


def _tc_body(p_ref, c_ref, W1_ref, b1_ref, W2_ref, b2_ref, W3_ref, b3_ref,
             out_ref):
    sums = p_ref[0]
    cnts = c_ref[0]
    for w in range(1, NW):
        sums = sums + p_ref[w]
        cnts = cnts + c_ref[w]
    pooled = sums / jnp.maximum(
        jnp.sum(cnts, axis=1, keepdims=True), 1.0)
    h = jnp.maximum(
        jnp.dot(pooled, W1_ref[...], preferred_element_type=jnp.float32)
        + b1_ref[...], 0.0)
    h = jnp.maximum(
        jnp.dot(h, W2_ref[...], preferred_element_type=jnp.float32)
        + b2_ref[...], 0.0)
    out_ref[...] = (
        jnp.dot(h, W3_ref[...], preferred_element_type=jnp.float32)
        + b3_ref[...])


def kernel(feat, segment_ids, W1, b1, W2, b2, W3, b3):
    ids = segment_ids.astype(jnp.int32)
    partials, counts = _sc_seg_sum(feat.reshape(N_NODES * D_FEAT), ids)
    pred = pl.pallas_call(
        _tc_body,
        out_shape=jax.ShapeDtypeStruct((NUM_GRAPHS, 1), jnp.float32),
    )(partials.reshape(NW, NUM_GRAPHS, D_FEAT),
      counts.reshape(NW, NUM_GRAPHS, 16),
      W1, b1.reshape(1, HIDDEN), W2, b2.reshape(1, HIDDEN),
      W3, b3.reshape(1, 1))
    return pred.reshape(NUM_GRAPHS)


# speedup vs baseline: 2.2490x; 1.0907x over previous
import functools

import jax
import jax.numpy as jnp
from jax import lax
from jax.experimental import pallas as pl
from jax.experimental.pallas import tpu as pltpu
from jax.experimental.pallas import tpu_sc as plsc

N_NODES = 100000
D_FEAT = 128
NUM_GRAPHS = 512
HIDDEN = 256

NC = 2
NS = 16
NW = NC * NS
ROWS_W = 3136
CHUNK = 192
NPAIRS = 9
NGROUPS = CHUNK // 16
NSLICE = D_FEAT // 16

_GDN = lax.GatherDimensionNumbers(
    offset_dims=(), collapsed_slice_dims=(0,), start_index_map=(0,))


def _lane_bcast(v, r):
    idx = jnp.full((16,), r, dtype=jnp.int32)
    return lax.gather(v, idx[:, None], _GDN, (1,),
                      mode=lax.GatherScatterMode.PROMISE_IN_BOUNDS)


def _sc_body(feat_hbm, ids_hbm, out_hbm, cnt_hbm,
             rowA, idsA, rowB, idsB, acc, cnt, semA, semB):
    cid = lax.axis_index("c")
    sid = lax.axis_index("s")
    wid = sid * NC + cid
    base = wid * ROWS_W
    end = jnp.minimum(base + ROWS_W, N_NODES)

    zero16 = jnp.zeros((16,), jnp.float32)
    ones16 = jnp.ones((16,), jnp.float32)
    iota16 = lax.iota(jnp.int32, 16)

    def start_chunk(k, rowbuf, idsbuf, sem):
        b = jnp.minimum(base + k * CHUNK, end - CHUNK)
        pltpu.async_copy(feat_hbm.at[pl.ds(b, CHUNK), :], rowbuf, sem)
        pltpu.async_copy(ids_hbm.at[pl.ds(b, CHUNK)], idsbuf, sem)

    def wait_chunk(rowbuf, idsbuf, sem):
        pltpu.make_async_copy(
            feat_hbm.at[pl.ds(0, CHUNK), :], rowbuf, sem).wait()
        pltpu.make_async_copy(
            ids_hbm.at[pl.ds(0, CHUNK)], idsbuf, sem).wait()

    start_chunk(0, rowA, idsA, semA)

    def zrow(i, carry):
        for j in range(NSLICE):
            acc[i, pl.ds(j * 16, 16)] = zero16
        cnt[i, pl.ds(0, 16)] = zero16
        return carry

    lax.fori_loop(0, NUM_GRAPHS, zrow, 0)

    def make_group_body(rowbuf, idsbuf):
        def group_body(g, carry):
            idv = idsbuf[pl.ds(g * 16, 16)]
            plsc.addupdate_scatter(cnt, [idv, iota16], ones16)
            prev = None
            for r in range(16):
                seg = _lane_bcast(idv, r)
                row = g * 16 + r
                xs = [rowbuf[row, pl.ds(j * 16, 16)] for j in range(NSLICE)]
                if prev is not None:
                    pseg, pxs = prev
                    for j in range(NSLICE):
                        plsc.addupdate_scatter(
                            acc, [pseg, iota16 + j * 16], pxs[j])
                prev = (seg, xs)
            pseg, pxs = prev
            for j in range(NSLICE):
                plsc.addupdate_scatter(acc, [pseg, iota16 + j * 16], pxs[j])
            return carry
        return group_body

    def process(k, p, rowbuf, idsbuf):
        b = jnp.minimum(base + k * CHUNK, end - CHUNK)
        gs = (p - b) // 16
        lax.fori_loop(gs, NGROUPS, make_group_body(rowbuf, idsbuf), 0)
        return b + CHUNK

    def pair_body(t, p):
        k0 = 2 * t
        start_chunk(k0 + 1, rowB, idsB, semB)
        wait_chunk(rowA, idsA, semA)
        p = process(k0, p, rowA, idsA)

        @pl.when(t < NPAIRS - 1)
        def _():
            start_chunk(k0 + 2, rowA, idsA, semA)

        wait_chunk(rowB, idsB, semB)
        return process(k0 + 1, p, rowB, idsB)

    lax.fori_loop(0, NPAIRS, pair_body, base)

    pltpu.sync_copy(acc, out_hbm.at[wid])
    pltpu.sync_copy(cnt, cnt_hbm.at[wid])


_sc_seg_sum = functools.partial(
    pl.kernel,
    out_type=[
        jax.ShapeDtypeStruct((NW, NUM_GRAPHS, D_FEAT), jnp.float32),
        jax.ShapeDtypeStruct((NW, NUM_GRAPHS, 16), jnp.float32),
    ],
    mesh=plsc.VectorSubcoreMesh(
        core_axis_name="c", subcore_axis_name="s",
        num_cores=NC, num_subcores=NS),
    compiler_params=pltpu.CompilerParams(
        needs_layout_passes=False, use_tc_tiling_on_sc=False),
    scratch_types=[
        pltpu.VMEM((CHUNK, D_FEAT), jnp.float32),
        pltpu.VMEM((CHUNK,), jnp.int32),
        pltpu.VMEM((CHUNK, D_FEAT), jnp.float32),
        pltpu.VMEM((CHUNK,), jnp.int32),
        pltpu.VMEM((NUM_GRAPHS, D_FEAT), jnp.float32),
        pltpu.VMEM((NUM_GRAPHS, 16), jnp.float32),
        pltpu.SemaphoreType.DMA,
        pltpu.SemaphoreType.DMA,
    ],
)(_sc_body)


def _tc_body(p_ref, c_ref, W1_ref, b1_ref, W2_ref, b2_ref, W3_ref, b3_ref,
             out_ref):
    sums = p_ref[0]
    cnts = c_ref[0]
    for w in range(1, NW):
        sums = sums + p_ref[w]
        cnts = cnts + c_ref[w]
    pooled = sums / jnp.maximum(
        jnp.sum(cnts, axis=1, keepdims=True), 1.0)
    h = jnp.maximum(
        jnp.dot(pooled, W1_ref[...], preferred_element_type=jnp.float32)
        + b1_ref[...], 0.0)
    h = jnp.maximum(
        jnp.dot(h, W2_ref[...], preferred_element_type=jnp.float32)
        + b2_ref[...], 0.0)
    out_ref[...] = (
        jnp.dot(h, W3_ref[...], preferred_element_type=jnp.float32)
        + b3_ref[...])


def kernel(feat, segment_ids, W1, b1, W2, b2, W3, b3):
    ids = segment_ids.astype(jnp.int32)
    partials, counts = _sc_seg_sum(feat, ids)
    pred = pl.pallas_call(
        _tc_body,
        out_shape=jax.ShapeDtypeStruct((NUM_GRAPHS, 1), jnp.float32),
    )(partials, counts,
      W1, b1.reshape(1, HIDDEN), W2, b2.reshape(1, HIDDEN),
      W3, b3.reshape(1, 1))
    return pred.reshape(NUM_GRAPHS)
